# baseline (speedup 1.0000x reference)
import jax
import jax.numpy as jnp
from jax.experimental import pallas as pl
from jax.experimental.pallas import tpu as pltpu

_B, _K, _H, _W = 16, 17, 80, 80


def _chunk_copies(refs, bufs, sems, c, slot):
    xi, xr, xs, xc, ti, tr, tc = refs
    xi_b, xr_b, xs_b, xc_b, ti_b, tr_b, tc_b = bufs
    sem = sems[slot]
    return (
        pltpu.make_async_copy(xi.at[c], xi_b.at[slot], sem),
        pltpu.make_async_copy(xr.at[c], xr_b.at[slot], sem),
        pltpu.make_async_copy(xs.at[c], xs_b.at[slot], sem),
        pltpu.make_async_copy(xc.at[c], xc_b.at[slot], sem),
        pltpu.make_async_copy(ti.at[c], ti_b.at[slot], sem),
        pltpu.make_async_copy(tr.at[c], tr_b.at[slot], sem),
        pltpu.make_async_copy(tc.at[c], tc_b.at[slot], sem),
    )


def _body(s2k_ref, xi, xr, xs, xc, ti, tr, tc, out_ref,
          xi_b, xr_b, xs_b, xc_b, ti_b, tr_b, tc_b, sem0, sem1):
    refs = (xi, xr, xs, xc, ti, tr, tc)
    bufs = (xi_b, xr_b, xs_b, xc_b, ti_b, tr_b, tc_b)
    sems = (sem0, sem1)

    ce_t = 0.0
    nsel_t = 0.0
    reg_t = 0.0
    sc_t = 0.0

    for cp in _chunk_copies(refs, bufs, sems, 0, 0):
        cp.start()

    for c in range(_B):
        slot = c % 2
        if c + 1 < _B:
            for cp in _chunk_copies(refs, bufs, sems, c + 1, (c + 1) % 2):
                cp.start()
        for cp in _chunk_copies(refs, bufs, sems, c, slot):
            cp.wait()

        tsum = ti_b[slot, _K]
        for k in range(_K):
            tsum = tsum + ti_b[slot, k]
        bce_mask = tsum > 0.5

        acc_per = jnp.zeros((_H, _W), jnp.float32)
        acc_reg = jnp.zeros((_H, _W), jnp.float32)
        acc_sc = jnp.zeros((_H, _W), jnp.float32)
        for k in range(_K):
            bt = ti_b[slot, k]
            x = xi_b[slot, k]
            acc_per += (jnp.maximum(x, 0.0) - x * bt
                        + jnp.log(1.0 + jnp.exp(-jnp.abs(x))))

            reg_mask = bt > 0.5
            d = ((xr_b[slot, k, 0] - tr_b[slot, k, 0]) ** 2
                 + (xr_b[slot, k, 1] - tr_b[slot, k, 1]) ** 2)
            norm = d * jax.lax.rsqrt(d + 1e-30)
            xss = xs_b[slot, k]
            lap = 0.694 + xss + norm * jnp.exp(-xss)
            acc_reg += jnp.where(reg_mask, lap, 0.0)

            sc = jnp.abs(xc_b[slot, k] - tc_b[slot, k] * s2k_ref[k])
            acc_sc += jnp.where(reg_mask, sc, 0.0)

        ce_t += jnp.sum(jnp.where(bce_mask, acc_per, 0.0))
        nsel_t += float(_K) * jnp.sum(bce_mask.astype(jnp.float32))
        reg_t += jnp.sum(acc_reg)
        sc_t += jnp.sum(acc_sc)

    out_ref[0] = ce_t
    out_ref[1] = nsel_t
    out_ref[2] = reg_t
    out_ref[3] = sc_t


def kernel(x_intensity, x_reg, x_spread, x_scale, t_intensity, t_reg,
           t_scale, scales_to_kp):
    s2k = jnp.broadcast_to(scales_to_kp.reshape(_K, 1, 1), (_K, 1, _W))

    any_spec = pl.BlockSpec(memory_space=pltpu.MemorySpace.HBM)
    sums = pl.pallas_call(
        _body,
        in_specs=[
            pl.BlockSpec((_K, 1, _W), lambda: (0, 0, 0)),
            any_spec, any_spec, any_spec, any_spec, any_spec, any_spec,
            any_spec,
        ],
        out_specs=pl.BlockSpec(memory_space=pltpu.SMEM),
        out_shape=jax.ShapeDtypeStruct((4,), jnp.float32),
        scratch_shapes=[
            pltpu.VMEM((2, _K, _H, _W), jnp.float32),
            pltpu.VMEM((2, _K, 2, _H, _W), jnp.float32),
            pltpu.VMEM((2, _K, _H, _W), jnp.float32),
            pltpu.VMEM((2, _K, _H, _W), jnp.float32),
            pltpu.VMEM((2, _K + 1, _H, _W), jnp.float32),
            pltpu.VMEM((2, _K, 2, _H, _W), jnp.float32),
            pltpu.VMEM((2, _K, _H, _W), jnp.float32),
            pltpu.SemaphoreType.DMA,
            pltpu.SemaphoreType.DMA,
        ],
    )(s2k, x_intensity, x_reg, x_spread, x_scale, t_intensity, t_reg,
      t_scale)

    ce_loss = sums[0] / sums[1]
    reg_loss = sums[2] / 1000.0 / _B
    scale_loss = sums[3] / 1000.0 / _B
    return (ce_loss, reg_loss, scale_loss)


# speedup vs baseline: 1.0316x; 1.0316x over previous
import jax
import jax.numpy as jnp
from jax.experimental import pallas as pl
from jax.experimental.pallas import tpu as pltpu

_B, _K, _H, _W = 16, 17, 80, 80
_BB = 2


def _body(s2k_ref, xi_ref, xr_ref, xs_ref, xc_ref, ti_ref, tr_ref, tc_ref,
          out_ref):
    step = pl.program_id(0)

    ce_part = 0.0
    nsel_part = 0.0
    reg_part = 0.0
    sc_part = 0.0
    for bb in range(_BB):
        tsum = ti_ref[bb, _K]
        for k in range(_K):
            tsum = tsum + ti_ref[bb, k]
        bce_mask = tsum > 0.5

        acc_per = jnp.zeros((_H, _W), jnp.float32)
        acc_reg = jnp.zeros((_H, _W), jnp.float32)
        acc_sc = jnp.zeros((_H, _W), jnp.float32)
        for k in range(_K):
            bt = ti_ref[bb, k]
            x = xi_ref[bb, k]
            acc_per += (jnp.maximum(x, 0.0) - x * bt
                        + jnp.log(1.0 + jnp.exp(-jnp.abs(x))))

            reg_mask = bt > 0.5
            d = ((xr_ref[bb, k, 0] - tr_ref[bb, k, 0]) ** 2
                 + (xr_ref[bb, k, 1] - tr_ref[bb, k, 1]) ** 2)
            norm = d * jax.lax.rsqrt(d + 1e-30)
            xs = xs_ref[bb, k]
            lap = 0.694 + xs + norm * jnp.exp(-xs)
            acc_reg += jnp.where(reg_mask, lap, 0.0)

            sc = jnp.abs(xc_ref[bb, k] - tc_ref[bb, k] * s2k_ref[k])
            acc_sc += jnp.where(reg_mask, sc, 0.0)

        ce_part += jnp.sum(jnp.where(bce_mask, acc_per, 0.0))
        nsel_part += float(_K) * jnp.sum(bce_mask.astype(jnp.float32))
        reg_part += jnp.sum(acc_reg)
        sc_part += jnp.sum(acc_sc)

    @pl.when(step == 0)
    def _():
        out_ref[0] = ce_part
        out_ref[1] = nsel_part
        out_ref[2] = reg_part
        out_ref[3] = sc_part

    @pl.when(step != 0)
    def _():
        out_ref[0] += ce_part
        out_ref[1] += nsel_part
        out_ref[2] += reg_part
        out_ref[3] += sc_part


def kernel(x_intensity, x_reg, x_spread, x_scale, t_intensity, t_reg,
           t_scale, scales_to_kp):
    s2k = jnp.broadcast_to(scales_to_kp.reshape(_K, 1, 1), (_K, 1, _W))

    sums = pl.pallas_call(
        _body,
        grid=(_B // _BB,),
        in_specs=[
            pl.BlockSpec((_K, 1, _W), lambda b: (0, 0, 0)),
            pl.BlockSpec((_BB, _K, _H, _W), lambda b: (b, 0, 0, 0)),
            pl.BlockSpec((_BB, _K, 2, _H, _W), lambda b: (b, 0, 0, 0, 0)),
            pl.BlockSpec((_BB, _K, _H, _W), lambda b: (b, 0, 0, 0)),
            pl.BlockSpec((_BB, _K, _H, _W), lambda b: (b, 0, 0, 0)),
            pl.BlockSpec((_BB, _K + 1, _H, _W), lambda b: (b, 0, 0, 0)),
            pl.BlockSpec((_BB, _K, 2, _H, _W), lambda b: (b, 0, 0, 0, 0)),
            pl.BlockSpec((_BB, _K, _H, _W), lambda b: (b, 0, 0, 0)),
        ],
        out_specs=pl.BlockSpec(memory_space=pltpu.SMEM),
        out_shape=jax.ShapeDtypeStruct((4,), jnp.float32),
    )(s2k, x_intensity, x_reg, x_spread, x_scale, t_intensity, t_reg,
      t_scale)

    ce_loss = sums[0] / sums[1]
    reg_loss = sums[2] / 1000.0 / _B
    scale_loss = sums[3] / 1000.0 / _B
    return (ce_loss, reg_loss, scale_loss)
